# baseline (speedup 1.0000x reference)
import numpy as np
import jax
import jax.numpy as jnp
from jax.experimental import pallas as pl
from jax.experimental.pallas import tpu as pltpu

_DIFFUSION_STEPS = 1000
_BETA_START = 0.0001
_BETA_END = 0.02


def _make_tables():
    betas = np.linspace(_BETA_START, _BETA_END, _DIFFUSION_STEPS, dtype=np.float32)
    alphas = (np.float32(1.0) - betas).astype(np.float32)
    alphas_cumprod = np.cumprod(alphas, dtype=np.float32)
    sqrt_acp = np.sqrt(alphas_cumprod).astype(np.float32)
    sqrt_omacp = np.sqrt((np.float32(1.0) - alphas_cumprod)).astype(np.float32)
    return sqrt_acp, sqrt_omacp


_SQRT_ACP, _SQRT_OMACP = _make_tables()

_LANES = 128


_SAMPLES_PER_BLOCK = 2


def _noise_body(ts_ref, a_tab_ref, b_tab_ref, x_ref, n_ref, o_ref, n_out_ref):
    i = pl.program_id(0)
    for s in range(_SAMPLES_PER_BLOCK):
        t = ts_ref[i * _SAMPLES_PER_BLOCK + s]
        a = a_tab_ref[t]
        b = b_tab_ref[t]
        nv = n_ref[s]
        o_ref[s] = a * x_ref[s] + b * nv
        n_out_ref[s] = nv


def kernel(clean_future, timesteps, noise):
    batch, ch, h, w = clean_future.shape

    spb = _SAMPLES_PER_BLOCK
    block = (spb, ch, h, w)
    grid_spec = pltpu.PrefetchScalarGridSpec(
        num_scalar_prefetch=3,
        grid=(batch // spb,),
        in_specs=[
            pl.BlockSpec(block, lambda i, *_: (i, 0, 0, 0)),
            pl.BlockSpec(block, lambda i, *_: (i, 0, 0, 0)),
        ],
        out_specs=[
            pl.BlockSpec(block, lambda i, *_: (i, 0, 0, 0)),
            pl.BlockSpec(block, lambda i, *_: (i, 0, 0, 0)),
        ],
    )

    out, n_out = pl.pallas_call(
        _noise_body,
        grid_spec=grid_spec,
        out_shape=[
            jax.ShapeDtypeStruct(clean_future.shape, jnp.float32),
            jax.ShapeDtypeStruct(clean_future.shape, jnp.float32),
        ],
    )(timesteps, jnp.asarray(_SQRT_ACP), jnp.asarray(_SQRT_OMACP), clean_future, noise)

    return out, n_out


# speedup vs baseline: 1.0840x; 1.0840x over previous
import numpy as np
import jax
import jax.numpy as jnp
from jax.experimental import pallas as pl
from jax.experimental.pallas import tpu as pltpu

_DIFFUSION_STEPS = 1000
_BETA_START = 0.0001
_BETA_END = 0.02


def _make_tables():
    betas = np.linspace(_BETA_START, _BETA_END, _DIFFUSION_STEPS, dtype=np.float32)
    alphas = (np.float32(1.0) - betas).astype(np.float32)
    alphas_cumprod = np.cumprod(alphas, dtype=np.float32)
    sqrt_acp = np.sqrt(alphas_cumprod).astype(np.float32)
    sqrt_omacp = np.sqrt((np.float32(1.0) - alphas_cumprod)).astype(np.float32)
    return sqrt_acp, sqrt_omacp


_SQRT_ACP, _SQRT_OMACP = _make_tables()

_LANES = 128


_SAMPLES_PER_BLOCK = 8


def _noise_body(ts_ref, a_tab_ref, b_tab_ref, x_ref, n_ref, o_ref, n_out_ref):
    i = pl.program_id(0)
    for s in range(_SAMPLES_PER_BLOCK):
        t = ts_ref[i * _SAMPLES_PER_BLOCK + s]
        a = a_tab_ref[t]
        b = b_tab_ref[t]
        nv = n_ref[s]
        o_ref[s] = a * x_ref[s] + b * nv
        n_out_ref[s] = nv


def kernel(clean_future, timesteps, noise):
    batch, ch, h, w = clean_future.shape

    spb = _SAMPLES_PER_BLOCK
    block = (spb, ch, h, w)
    grid_spec = pltpu.PrefetchScalarGridSpec(
        num_scalar_prefetch=3,
        grid=(batch // spb,),
        in_specs=[
            pl.BlockSpec(block, lambda i, *_: (i, 0, 0, 0)),
            pl.BlockSpec(block, lambda i, *_: (i, 0, 0, 0)),
        ],
        out_specs=[
            pl.BlockSpec(block, lambda i, *_: (i, 0, 0, 0)),
            pl.BlockSpec(block, lambda i, *_: (i, 0, 0, 0)),
        ],
    )

    out, n_out = pl.pallas_call(
        _noise_body,
        grid_spec=grid_spec,
        out_shape=[
            jax.ShapeDtypeStruct(clean_future.shape, jnp.float32),
            jax.ShapeDtypeStruct(clean_future.shape, jnp.float32),
        ],
    )(timesteps, jnp.asarray(_SQRT_ACP), jnp.asarray(_SQRT_OMACP), clean_future, noise)

    return out, n_out
